# Initial kernel scaffold; baseline (speedup 1.0000x reference)
import jax
import jax.numpy as jnp
from jax.experimental import pallas as pl


def kernel(x, edge_index, W, b):
    raise NotImplementedError("write your pallas kernel here")


# speedup vs baseline: 4.9550x; 4.9550x over previous
import functools

import jax
import jax.numpy as jnp
from jax import lax
from jax.experimental import pallas as pl
from jax.experimental.pallas import tpu as pltpu
from jax.experimental.pallas import tpu_sc as plsc

MIN_NORM = 1e-15
PROJ_EPS = 4e-3
MAXNORM = 1.0 - PROJ_EPS

NC = 2
NT = 16
BATCH = 128


def _artanh(v):
    v = jnp.clip(v, -1.0 + 1e-7, 1.0 - 1e-7)
    return 0.5 * jnp.log((1.0 + v) / (1.0 - v))


def _row_norm(v):
    return jnp.maximum(jnp.sqrt(jnp.sum(v * v, axis=1, keepdims=True)), MIN_NORM)


def _proj(v):
    n = _row_norm(v)
    return jnp.where(n > MAXNORM, v / n * MAXNORM, v)


def _expmap0(u):
    un = _row_norm(u)
    return jnp.tanh(un) * u / un


def _logmap0(p):
    pn = _row_norm(p)
    return _artanh(pn) * p / pn


def _stage1_body(x_ref, wt_ref, hb_ref, out_ref):
    xb = x_ref[...]
    x_norm = _row_norm(xb)
    mx = jnp.dot(xb, wt_ref[...], preferred_element_type=jnp.float32)
    mx_norm = _row_norm(mx)
    res_c = jnp.tanh(mx_norm / x_norm * _artanh(x_norm)) * mx / mx_norm
    allzero = jnp.all(mx == 0.0, axis=1, keepdims=True)
    res = _proj(jnp.where(allzero, 0.0, res_c))
    y = hb_ref[...]
    x2 = jnp.sum(res * res, axis=1, keepdims=True)
    y2 = jnp.sum(y * y, axis=1, keepdims=True)
    xy = jnp.sum(res * y, axis=1, keepdims=True)
    num = (1.0 + 2.0 * xy + y2) * res + (1.0 - x2) * y
    den = 1.0 + 2.0 * xy + x2 * y2
    res = _proj(num / jnp.maximum(den, MIN_NORM))
    xt = _logmap0(res)
    half = xt.shape[1] // 2
    out_ref[0] = xt[:, :half]
    out_ref[1] = xt[:, half:]


def _stage3_body(lo_ref, hi_ref, out_ref):
    st = jnp.concatenate([lo_ref[...], hi_ref[...]], axis=1)
    h = _proj(_expmap0(st))
    ht = jnp.maximum(_logmap0(h), 0.0)
    out_ref[...] = _proj(_expmap0(ht))


def _make_scatter_kernel(n, half, rpt, acc_rows):
    mesh = plsc.VectorSubcoreMesh(
        core_axis_name="c", subcore_axis_name="s", num_cores=NC, num_subcores=NT
    )
    zrows = acc_rows // NT
    wrows = zrows
    last_rows = n - (NT - 1) * wrows
    assert 0 < last_rows <= wrows and last_rows % 8 == 0

    @functools.partial(
        pl.kernel,
        out_type=jax.ShapeDtypeStruct((NC * n, half), jnp.float32),
        mesh=mesh,
        scratch_types=[
            pltpu.VMEM((rpt, BATCH), jnp.int32),
            pltpu.VMEM((rpt, BATCH), jnp.int32),
            pltpu.VMEM((BATCH, half), jnp.float32),
            pltpu.VMEM_SHARED((acc_rows, half), jnp.float32),
            pltpu.SemaphoreType.DMA,
        ],
    )
    def scatter_k(table_hbm, srcs_hbm, dsts_hbm, zeros_hbm, out_hbm,
                  src_v, dst_v, rows_v, acc_sh, sem):
        c = lax.axis_index("c")
        s = lax.axis_index("s")
        pltpu.sync_copy(zeros_hbm, acc_sh.at[pl.ds(s * zrows, zrows)])
        pltpu.sync_copy(srcs_hbm.at[pl.ds((c * NT + s) * rpt, rpt)], src_v)
        pltpu.sync_copy(dsts_hbm.at[pl.ds(s * rpt, rpt)], dst_v)
        plsc.subcore_barrier()

        @pl.loop(0, rpt)
        def _edge_block(j):
            pltpu.async_copy(table_hbm.at[src_v.at[j]], rows_v, sem).wait()
            pltpu.sync_copy(rows_v, acc_sh.at[dst_v.at[j]], add=True)

        plsc.subcore_barrier()

        @pl.when(s < NT - 1)
        def _wb_full():
            pltpu.sync_copy(
                acc_sh.at[pl.ds(s * wrows, wrows)],
                out_hbm.at[pl.ds(c * n + s * wrows, wrows)],
            )

        @pl.when(s == NT - 1)
        def _wb_last():
            pltpu.sync_copy(
                acc_sh.at[pl.ds((NT - 1) * wrows, last_rows)],
                out_hbm.at[pl.ds(c * n + (NT - 1) * wrows, last_rows)],
            )

    return scatter_k


def kernel(x, edge_index, W, b):
    n, d = x.shape
    half = d // 2
    e = edge_index.shape[1]
    rpt = -(-e // (NT * BATCH))
    rpt = -(-rpt // 8) * 8
    e_pad = rpt * NT * BATCH
    acc_rows = -(-(n + 1) // 128) * 128

    bias = b.reshape(1, -1).astype(jnp.float32)
    bn = jnp.maximum(jnp.linalg.norm(bias, axis=-1, keepdims=True), MIN_NORM)
    hb = jnp.tanh(bn) * bias / bn
    hbn = jnp.maximum(jnp.linalg.norm(hb, axis=-1, keepdims=True), MIN_NORM)
    hb = jnp.where(hbn > MAXNORM, hb / hbn * MAXNORM, hb)

    src = edge_index[0].astype(jnp.int32)
    dst = edge_index[1].astype(jnp.int32)
    pad = e_pad - e
    fill = jnp.arange(pad, dtype=jnp.int32)
    src_p = jnp.concatenate([src, fill % n])
    dst_p = jnp.concatenate([dst, n + (fill % (acc_rows - n))])
    srcs = jnp.concatenate([src_p, src_p + n]).reshape(NC * e_pad // BATCH, BATCH)
    dsts = dst_p.reshape(e_pad // BATCH, BATCH)
    zeros = jnp.zeros((acc_rows // NT, half), jnp.float32)

    r1 = 400
    xt2 = pl.pallas_call(
        _stage1_body,
        grid=(n // r1,),
        in_specs=[
            pl.BlockSpec((r1, d), lambda i: (i, 0)),
            pl.BlockSpec((d, d), lambda i: (0, 0)),
            pl.BlockSpec((1, d), lambda i: (0, 0)),
        ],
        out_specs=pl.BlockSpec((2, r1, half), lambda i: (0, i, 0)),
        out_shape=jax.ShapeDtypeStruct((2, n, half), jnp.float32),
    )(x, W.T, hb)
    table = xt2.reshape(2 * n, half)

    sup = _make_scatter_kernel(n, half, rpt, acc_rows)(table, srcs, dsts, zeros)

    r3 = 400
    nb = n // r3
    out = pl.pallas_call(
        _stage3_body,
        grid=(nb,),
        in_specs=[
            pl.BlockSpec((r3, half), lambda i: (i, 0)),
            pl.BlockSpec((r3, half), lambda i: (i + nb, 0)),
        ],
        out_specs=pl.BlockSpec((r3, d), lambda i: (i, 0)),
        out_shape=jax.ShapeDtypeStruct((n, d), jnp.float32),
    )(sup, sup)
    return out


# speedup vs baseline: 6.5873x; 1.3294x over previous
import functools

import jax
import jax.numpy as jnp
from jax import lax
from jax.experimental import pallas as pl
from jax.experimental.pallas import tpu as pltpu
from jax.experimental.pallas import tpu_sc as plsc

MIN_NORM = 1e-15
PROJ_EPS = 4e-3
MAXNORM = 1.0 - PROJ_EPS

NC = 2
NT = 16
BATCH = 128


def _artanh(v):
    v = jnp.clip(v, -1.0 + 1e-7, 1.0 - 1e-7)
    return 0.5 * jnp.log((1.0 + v) / (1.0 - v))


def _row_norm(v):
    return jnp.maximum(jnp.sqrt(jnp.sum(v * v, axis=1, keepdims=True)), MIN_NORM)


def _proj(v):
    n = _row_norm(v)
    return jnp.where(n > MAXNORM, v / n * MAXNORM, v)


def _expmap0(u):
    un = _row_norm(u)
    return jnp.tanh(un) * u / un


def _logmap0(p):
    pn = _row_norm(p)
    return _artanh(pn) * p / pn


def _stage1_body(x_ref, wt_ref, hb_ref, out_ref):
    xb = x_ref[...]
    x_norm = _row_norm(xb)
    mx = jnp.dot(xb, wt_ref[...], preferred_element_type=jnp.float32)
    mx_norm = _row_norm(mx)
    res_c = jnp.tanh(mx_norm / x_norm * _artanh(x_norm)) * mx / mx_norm
    allzero = jnp.all(mx == 0.0, axis=1, keepdims=True)
    res = _proj(jnp.where(allzero, 0.0, res_c))
    y = hb_ref[...]
    x2 = jnp.sum(res * res, axis=1, keepdims=True)
    y2 = jnp.sum(y * y, axis=1, keepdims=True)
    xy = jnp.sum(res * y, axis=1, keepdims=True)
    num = (1.0 + 2.0 * xy + y2) * res + (1.0 - x2) * y
    den = 1.0 + 2.0 * xy + x2 * y2
    res = _proj(num / jnp.maximum(den, MIN_NORM))
    xt = _logmap0(res)
    half = xt.shape[1] // 2
    out_ref[0] = xt[:, :half]
    out_ref[1] = xt[:, half:]


def _stage3_body(lo_ref, hi_ref, out_ref):
    st = jnp.concatenate([lo_ref[...], hi_ref[...]], axis=1)
    h = _proj(_expmap0(st))
    ht = jnp.maximum(_logmap0(h), 0.0)
    out_ref[...] = _proj(_expmap0(ht))


def _make_scatter_kernel(n, half, rpt, acc_rows):
    mesh = plsc.VectorSubcoreMesh(
        core_axis_name="c", subcore_axis_name="s", num_cores=NC, num_subcores=NT
    )
    zrows = acc_rows // NT
    wrows = zrows
    last_rows = n - (NT - 1) * wrows
    assert 0 < last_rows <= wrows and last_rows % 8 == 0

    @functools.partial(
        pl.kernel,
        out_type=jax.ShapeDtypeStruct((NC * n, half), jnp.float32),
        mesh=mesh,
        scratch_types=[
            pltpu.VMEM((rpt // 2, BATCH), jnp.int32),
            pltpu.VMEM((rpt // 2, BATCH), jnp.int32),
            pltpu.VMEM((BATCH, half), jnp.float32),
            pltpu.VMEM((BATCH, half), jnp.float32),
            pltpu.VMEM_SHARED((acc_rows, half), jnp.float32),
            pltpu.SemaphoreType.DMA,
            pltpu.SemaphoreType.DMA,
        ],
    )
    def scatter_k(table_hbm, srcs_hbm, dsts_hbm, zeros_hbm, out_hbm,
                  src_v, dst_v, buf0, buf1, acc_sh, sem0, sem1):
        c = lax.axis_index("c")
        s = lax.axis_index("s")
        pltpu.sync_copy(zeros_hbm, acc_sh.at[pl.ds(s * zrows, zrows)])
        plsc.subcore_barrier()

        ph = rpt // 2
        for phase in range(2):
            pltpu.sync_copy(
                srcs_hbm.at[pl.ds((c * NT + s) * rpt + phase * ph, ph)], src_v
            )
            pltpu.sync_copy(dsts_hbm.at[pl.ds(s * rpt + phase * ph, ph)], dst_v)
            pltpu.async_copy(table_hbm.at[src_v.at[0]], buf0, sem0)

            @pl.loop(0, ph, step=2)
            def _edge_block(j):
                pltpu.async_copy(table_hbm.at[src_v.at[j + 1]], buf1, sem1)
                pltpu.make_async_copy(table_hbm.at[src_v.at[j]], buf0, sem0).wait()
                pltpu.sync_copy(buf0, acc_sh.at[dst_v.at[j]], add=True)

                @pl.when(j + 2 < ph)
                def _next():
                    pltpu.async_copy(table_hbm.at[src_v.at[j + 2]], buf0, sem0)

                pltpu.make_async_copy(table_hbm.at[src_v.at[j + 1]], buf1, sem1).wait()
                pltpu.sync_copy(buf1, acc_sh.at[dst_v.at[j + 1]], add=True)

        plsc.subcore_barrier()

        @pl.when(s < NT - 1)
        def _wb_full():
            pltpu.sync_copy(
                acc_sh.at[pl.ds(s * wrows, wrows)],
                out_hbm.at[pl.ds(c * n + s * wrows, wrows)],
            )

        @pl.when(s == NT - 1)
        def _wb_last():
            pltpu.sync_copy(
                acc_sh.at[pl.ds((NT - 1) * wrows, last_rows)],
                out_hbm.at[pl.ds(c * n + (NT - 1) * wrows, last_rows)],
            )

    return scatter_k


def kernel(x, edge_index, W, b):
    n, d = x.shape
    half = d // 2
    e = edge_index.shape[1]
    rpt = -(-e // (NT * BATCH))
    rpt = -(-rpt // 8) * 8
    e_pad = rpt * NT * BATCH
    acc_rows = -(-(n + 1) // 128) * 128

    bias = b.reshape(1, -1).astype(jnp.float32)
    bn = jnp.maximum(jnp.linalg.norm(bias, axis=-1, keepdims=True), MIN_NORM)
    hb = jnp.tanh(bn) * bias / bn
    hbn = jnp.maximum(jnp.linalg.norm(hb, axis=-1, keepdims=True), MIN_NORM)
    hb = jnp.where(hbn > MAXNORM, hb / hbn * MAXNORM, hb)

    src = edge_index[0].astype(jnp.int32)
    dst = edge_index[1].astype(jnp.int32)
    pad = e_pad - e
    fill = jnp.arange(pad, dtype=jnp.int32)
    src_p = jnp.concatenate([src, fill % n])
    dst_p = jnp.concatenate([dst, n + (fill % (acc_rows - n))])
    srcs = jnp.concatenate([src_p, src_p + n]).reshape(NC * e_pad // BATCH, BATCH)
    dsts = dst_p.reshape(e_pad // BATCH, BATCH)
    zeros = jnp.zeros((acc_rows // NT, half), jnp.float32)

    r1 = 400
    xt2 = pl.pallas_call(
        _stage1_body,
        grid=(n // r1,),
        in_specs=[
            pl.BlockSpec((r1, d), lambda i: (i, 0)),
            pl.BlockSpec((d, d), lambda i: (0, 0)),
            pl.BlockSpec((1, d), lambda i: (0, 0)),
        ],
        out_specs=pl.BlockSpec((2, r1, half), lambda i: (0, i, 0)),
        out_shape=jax.ShapeDtypeStruct((2, n, half), jnp.float32),
    )(x, W.T, hb)
    table = xt2.reshape(2 * n, half)

    sup = _make_scatter_kernel(n, half, rpt, acc_rows)(table, srcs, dsts, zeros)

    r3 = 400
    nb = n // r3
    out = pl.pallas_call(
        _stage3_body,
        grid=(nb,),
        in_specs=[
            pl.BlockSpec((r3, half), lambda i: (i, 0)),
            pl.BlockSpec((r3, half), lambda i: (i + nb, 0)),
        ],
        out_specs=pl.BlockSpec((r3, d), lambda i: (i, 0)),
        out_shape=jax.ShapeDtypeStruct((n, d), jnp.float32),
    )(sup, sup)
    return out


# speedup vs baseline: 6.5972x; 1.0015x over previous
import functools

import jax
import jax.numpy as jnp
from jax import lax
from jax.experimental import pallas as pl
from jax.experimental.pallas import tpu as pltpu
from jax.experimental.pallas import tpu_sc as plsc

MIN_NORM = 1e-15
PROJ_EPS = 4e-3
MAXNORM = 1.0 - PROJ_EPS

NC = 2
NT = 16
BATCH = 128


def _artanh(v):
    v = jnp.clip(v, -1.0 + 1e-7, 1.0 - 1e-7)
    return 0.5 * jnp.log((1.0 + v) / (1.0 - v))


def _row_norm(v):
    return jnp.maximum(jnp.sqrt(jnp.sum(v * v, axis=1, keepdims=True)), MIN_NORM)


def _proj(v):
    n = _row_norm(v)
    return jnp.where(n > MAXNORM, v / n * MAXNORM, v)


def _expmap0(u):
    un = _row_norm(u)
    return jnp.tanh(un) * u / un


def _logmap0(p):
    pn = _row_norm(p)
    return _artanh(pn) * p / pn


def _stage1_body(x_ref, wt_ref, hb_ref, out_ref):
    xb = x_ref[...]
    x_norm = _row_norm(xb)
    mx = jnp.dot(xb, wt_ref[...], preferred_element_type=jnp.float32)
    mx_norm = _row_norm(mx)
    res_c = jnp.tanh(mx_norm / x_norm * _artanh(x_norm)) * mx / mx_norm
    allzero = jnp.all(mx == 0.0, axis=1, keepdims=True)
    res = _proj(jnp.where(allzero, 0.0, res_c))
    y = hb_ref[...]
    x2 = jnp.sum(res * res, axis=1, keepdims=True)
    y2 = jnp.sum(y * y, axis=1, keepdims=True)
    xy = jnp.sum(res * y, axis=1, keepdims=True)
    num = (1.0 + 2.0 * xy + y2) * res + (1.0 - x2) * y
    den = 1.0 + 2.0 * xy + x2 * y2
    res = _proj(num / jnp.maximum(den, MIN_NORM))
    xt = _logmap0(res)
    half = xt.shape[1] // 2
    out_ref[0] = xt[:, :half]
    out_ref[1] = xt[:, half:]


def _stage3_body(lo_ref, hi_ref, out_ref):
    st = jnp.concatenate([lo_ref[...], hi_ref[...]], axis=1)
    h = _proj(_expmap0(st))
    ht = jnp.maximum(_logmap0(h), 0.0)
    out_ref[...] = _proj(_expmap0(ht))


def _make_scatter_kernel(n, half, rpt, acc_rows):
    mesh = plsc.VectorSubcoreMesh(
        core_axis_name="c", subcore_axis_name="s", num_cores=NC, num_subcores=NT
    )
    zrows = acc_rows // NT
    wrows = zrows
    last_rows = n - (NT - 1) * wrows
    assert 0 < last_rows <= wrows and last_rows % 8 == 0

    @functools.partial(
        pl.kernel,
        out_type=jax.ShapeDtypeStruct((NC * n, half), jnp.float32),
        mesh=mesh,
        scratch_types=[
            pltpu.VMEM((rpt // 2, BATCH), jnp.int32),
            pltpu.VMEM((rpt // 2, BATCH), jnp.int32),
            pltpu.VMEM((BATCH, half), jnp.float32),
            pltpu.VMEM((BATCH, half), jnp.float32),
            pltpu.VMEM_SHARED((acc_rows, half), jnp.float32),
            pltpu.SemaphoreType.DMA,
            pltpu.SemaphoreType.DMA,
        ],
    )
    def scatter_k(table_hbm, srcs_hbm, dsts_hbm, zeros_hbm, out_hbm,
                  src_v, dst_v, buf0, buf1, acc_sh, sem0, sem1):
        c = lax.axis_index("c")
        s = lax.axis_index("s")
        pltpu.sync_copy(zeros_hbm, acc_sh.at[pl.ds(s * zrows, zrows)])
        plsc.subcore_barrier()

        ph = rpt // 2
        for phase in range(2):
            pltpu.sync_copy(
                srcs_hbm.at[pl.ds((c * NT + s) * rpt + phase * ph, ph)], src_v
            )
            pltpu.sync_copy(dsts_hbm.at[pl.ds(s * rpt + phase * ph, ph)], dst_v)
            pltpu.async_copy(table_hbm.at[src_v.at[0]], buf0, sem0)

            @pl.loop(0, ph, step=2, unroll=4)
            def _edge_block(j):
                pltpu.async_copy(table_hbm.at[src_v.at[j + 1]], buf1, sem1)
                pltpu.make_async_copy(table_hbm.at[src_v.at[j]], buf0, sem0).wait()
                pltpu.sync_copy(buf0, acc_sh.at[dst_v.at[j]], add=True)

                @pl.when(j + 2 < ph)
                def _next():
                    pltpu.async_copy(table_hbm.at[src_v.at[j + 2]], buf0, sem0)

                pltpu.make_async_copy(table_hbm.at[src_v.at[j + 1]], buf1, sem1).wait()
                pltpu.sync_copy(buf1, acc_sh.at[dst_v.at[j + 1]], add=True)

        plsc.subcore_barrier()

        @pl.when(s < NT - 1)
        def _wb_full():
            pltpu.sync_copy(
                acc_sh.at[pl.ds(s * wrows, wrows)],
                out_hbm.at[pl.ds(c * n + s * wrows, wrows)],
            )

        @pl.when(s == NT - 1)
        def _wb_last():
            pltpu.sync_copy(
                acc_sh.at[pl.ds((NT - 1) * wrows, last_rows)],
                out_hbm.at[pl.ds(c * n + (NT - 1) * wrows, last_rows)],
            )

    return scatter_k


def kernel(x, edge_index, W, b):
    n, d = x.shape
    half = d // 2
    e = edge_index.shape[1]
    rpt = -(-e // (NT * BATCH))
    rpt = -(-rpt // 8) * 8
    e_pad = rpt * NT * BATCH
    acc_rows = -(-(n + 1) // 128) * 128

    bias = b.reshape(1, -1).astype(jnp.float32)
    bn = jnp.maximum(jnp.linalg.norm(bias, axis=-1, keepdims=True), MIN_NORM)
    hb = jnp.tanh(bn) * bias / bn
    hbn = jnp.maximum(jnp.linalg.norm(hb, axis=-1, keepdims=True), MIN_NORM)
    hb = jnp.where(hbn > MAXNORM, hb / hbn * MAXNORM, hb)

    src = edge_index[0].astype(jnp.int32)
    dst = edge_index[1].astype(jnp.int32)
    pad = e_pad - e
    fill = jnp.arange(pad, dtype=jnp.int32)
    src_p = jnp.concatenate([src, fill % n])
    dst_p = jnp.concatenate([dst, n + (fill % (acc_rows - n))])
    srcs = jnp.concatenate([src_p, src_p + n]).reshape(NC * e_pad // BATCH, BATCH)
    dsts = dst_p.reshape(e_pad // BATCH, BATCH)
    zeros = jnp.zeros((acc_rows // NT, half), jnp.float32)

    r1 = 400
    xt2 = pl.pallas_call(
        _stage1_body,
        grid=(n // r1,),
        in_specs=[
            pl.BlockSpec((r1, d), lambda i: (i, 0)),
            pl.BlockSpec((d, d), lambda i: (0, 0)),
            pl.BlockSpec((1, d), lambda i: (0, 0)),
        ],
        out_specs=pl.BlockSpec((2, r1, half), lambda i: (0, i, 0)),
        out_shape=jax.ShapeDtypeStruct((2, n, half), jnp.float32),
    )(x, W.T, hb)
    table = xt2.reshape(2 * n, half)

    sup = _make_scatter_kernel(n, half, rpt, acc_rows)(table, srcs, dsts, zeros)

    r3 = 400
    nb = n // r3
    out = pl.pallas_call(
        _stage3_body,
        grid=(nb,),
        in_specs=[
            pl.BlockSpec((r3, half), lambda i: (i, 0)),
            pl.BlockSpec((r3, half), lambda i: (i + nb, 0)),
        ],
        out_specs=pl.BlockSpec((r3, d), lambda i: (i, 0)),
        out_shape=jax.ShapeDtypeStruct((n, d), jnp.float32),
    )(sup, sup)
    return out


# speedup vs baseline: 6.9779x; 1.0577x over previous
import functools

import jax
import jax.numpy as jnp
from jax import lax
from jax.experimental import pallas as pl
from jax.experimental.pallas import tpu as pltpu
from jax.experimental.pallas import tpu_sc as plsc

MIN_NORM = 1e-15
PROJ_EPS = 4e-3
MAXNORM = 1.0 - PROJ_EPS

NC = 2
NT = 16
BATCH = 128


def _artanh(v):
    v = jnp.clip(v, -1.0 + 1e-7, 1.0 - 1e-7)
    return 0.5 * jnp.log((1.0 + v) / (1.0 - v))


def _row_norm(v):
    return jnp.maximum(jnp.sqrt(jnp.sum(v * v, axis=1, keepdims=True)), MIN_NORM)


def _proj(v):
    n = _row_norm(v)
    return jnp.where(n > MAXNORM, v / n * MAXNORM, v)


def _expmap0(u):
    un = _row_norm(u)
    return jnp.tanh(un) * u / un


def _logmap0(p):
    pn = _row_norm(p)
    return _artanh(pn) * p / pn


def _proj_scale(norm_raw):
    return jnp.where(
        norm_raw > MAXNORM, MAXNORM / jnp.maximum(norm_raw, MIN_NORM), 1.0
    )


def _stage1_body(x_ref, w_ref, b_ref, out_ref):
    xb = x_ref[...]
    xn = jnp.maximum(
        jnp.sqrt(jnp.sum(xb * xb, axis=1, keepdims=True)), MIN_NORM
    )
    mx = lax.dot_general(
        xb, w_ref[...], (((1,), (1,)), ((), ())),
        preferred_element_type=jnp.float32,
    )
    m2 = jnp.sum(mx * mx, axis=1, keepdims=True)
    mn_raw = jnp.sqrt(m2)
    mn = jnp.maximum(mn_raw, MIN_NORM)
    rc = jnp.tanh(mn / xn * _artanh(xn)) / mn
    rn_raw = rc * mn_raw
    rc2 = rc * _proj_scale(rn_raw)
    rn = rn_raw * _proj_scale(rn_raw)
    bb = b_ref[...]
    bn = jnp.maximum(
        jnp.sqrt(jnp.sum(bb * bb, axis=1, keepdims=True)), MIN_NORM
    )
    hb = jnp.tanh(bn) * bb / bn
    hbn_raw = jnp.sqrt(jnp.sum(hb * hb, axis=1, keepdims=True))
    hb = hb * _proj_scale(hbn_raw)
    y2 = jnp.sum(hb * hb, axis=1, keepdims=True)
    xy = rc2 * jnp.sum(mx * hb, axis=1, keepdims=True)
    x2 = rn * rn
    den = jnp.maximum(1.0 + 2.0 * xy + x2 * y2, MIN_NORM)
    num = ((1.0 + 2.0 * xy + y2) * rc2) * mx + (1.0 - x2) * hb
    q_raw = jnp.sqrt(jnp.sum(num * num, axis=1, keepdims=True)) / den
    p2 = _proj_scale(q_raw)
    pn = jnp.maximum(q_raw * p2, MIN_NORM)
    xt = num * ((p2 / den) * (_artanh(pn) / pn))
    half = xt.shape[1] // 2
    out_ref[0] = xt[:, :half]
    out_ref[1] = xt[:, half:]


def _stage3_body(lo_ref, hi_ref, out_ref):
    u = jnp.concatenate([lo_ref[...], hi_ref[...]], axis=1)
    u2 = jnp.sum(u * u, axis=1, keepdims=True)
    un_raw = jnp.sqrt(u2)
    un = jnp.maximum(un_raw, MIN_NORM)
    a = jnp.tanh(un) / un
    hn_raw = a * un_raw
    a2 = a * _proj_scale(hn_raw)
    hn = jnp.maximum(hn_raw * _proj_scale(hn_raw), MIN_NORM)
    g = a2 * (_artanh(hn) / hn)
    v = jnp.maximum(u, 0.0)
    vn_raw = jnp.sqrt(jnp.sum(v * v, axis=1, keepdims=True)) * g
    vn = jnp.maximum(vn_raw, MIN_NORM)
    f = jnp.tanh(vn) / vn
    h2_raw = f * vn_raw
    out_ref[...] = v * (g * f * _proj_scale(h2_raw))


def _make_scatter_kernel(n, half, rpt, acc_rows):
    mesh = plsc.VectorSubcoreMesh(
        core_axis_name="c", subcore_axis_name="s", num_cores=NC, num_subcores=NT
    )
    zrows = acc_rows // NT
    wrows = zrows
    last_rows = n - (NT - 1) * wrows
    assert 0 < last_rows <= wrows and last_rows % 8 == 0

    @functools.partial(
        pl.kernel,
        out_type=jax.ShapeDtypeStruct((NC * n, half), jnp.float32),
        mesh=mesh,
        scratch_types=[
            pltpu.VMEM((rpt // 2, BATCH), jnp.int32),
            pltpu.VMEM((rpt // 2, BATCH), jnp.int32),
            pltpu.VMEM((BATCH, half), jnp.float32),
            pltpu.VMEM((BATCH, half), jnp.float32),
            pltpu.VMEM_SHARED((acc_rows, half), jnp.float32),
            pltpu.SemaphoreType.DMA,
            pltpu.SemaphoreType.DMA,
        ],
    )
    def scatter_k(table_hbm, srcs_hbm, dsts_hbm, zeros_hbm, out_hbm,
                  src_v, dst_v, buf0, buf1, acc_sh, sem0, sem1):
        c = lax.axis_index("c")
        s = lax.axis_index("s")
        pltpu.sync_copy(zeros_hbm, acc_sh.at[pl.ds(s * zrows, zrows)])
        plsc.subcore_barrier()

        ph = rpt // 2
        for phase in range(2):
            pltpu.sync_copy(
                srcs_hbm.at[pl.ds((c * NT + s) * rpt + phase * ph, ph)], src_v
            )
            pltpu.sync_copy(dsts_hbm.at[pl.ds(s * rpt + phase * ph, ph)], dst_v)
            pltpu.async_copy(table_hbm.at[src_v.at[0]], buf0, sem0)

            @pl.loop(0, ph, step=2, unroll=4)
            def _edge_block(j):
                pltpu.async_copy(table_hbm.at[src_v.at[j + 1]], buf1, sem1)
                pltpu.make_async_copy(table_hbm.at[src_v.at[j]], buf0, sem0).wait()
                pltpu.sync_copy(buf0, acc_sh.at[dst_v.at[j]], add=True)

                @pl.when(j + 2 < ph)
                def _next():
                    pltpu.async_copy(table_hbm.at[src_v.at[j + 2]], buf0, sem0)

                pltpu.make_async_copy(table_hbm.at[src_v.at[j + 1]], buf1, sem1).wait()
                pltpu.sync_copy(buf1, acc_sh.at[dst_v.at[j + 1]], add=True)

        plsc.subcore_barrier()

        @pl.when(s < NT - 1)
        def _wb_full():
            pltpu.sync_copy(
                acc_sh.at[pl.ds(s * wrows, wrows)],
                out_hbm.at[pl.ds(c * n + s * wrows, wrows)],
            )

        @pl.when(s == NT - 1)
        def _wb_last():
            pltpu.sync_copy(
                acc_sh.at[pl.ds((NT - 1) * wrows, last_rows)],
                out_hbm.at[pl.ds(c * n + (NT - 1) * wrows, last_rows)],
            )

    return scatter_k


def kernel(x, edge_index, W, b):
    n, d = x.shape
    half = d // 2
    e = edge_index.shape[1]
    rpt = -(-e // (NT * BATCH))
    rpt = -(-rpt // 8) * 8
    e_pad = rpt * NT * BATCH
    acc_rows = -(-(n + 1) // 128) * 128

    src = edge_index[0].astype(jnp.int32)
    dst = edge_index[1].astype(jnp.int32)
    pad = e_pad - e
    fill = jnp.arange(pad, dtype=jnp.int32)
    src_p = jnp.concatenate([src, fill % n])
    dst_p = jnp.concatenate([dst, n + (fill % (acc_rows - n))])
    srcs = jnp.concatenate([src_p, src_p + n]).reshape(NC * e_pad // BATCH, BATCH)
    dsts = dst_p.reshape(e_pad // BATCH, BATCH)
    zeros = jnp.zeros((acc_rows // NT, half), jnp.float32)

    r1 = 400
    xt2 = pl.pallas_call(
        _stage1_body,
        grid=(n // r1,),
        in_specs=[
            pl.BlockSpec((r1, d), lambda i: (i, 0)),
            pl.BlockSpec((d, d), lambda i: (0, 0)),
            pl.BlockSpec((1, d), lambda i: (0, 0)),
        ],
        out_specs=pl.BlockSpec((2, r1, half), lambda i: (0, i, 0)),
        out_shape=jax.ShapeDtypeStruct((2, n, half), jnp.float32),
    )(x, W, b.reshape(1, -1))
    table = xt2.reshape(2 * n, half)

    sup = _make_scatter_kernel(n, half, rpt, acc_rows)(table, srcs, dsts, zeros)

    r3 = 400
    nb = n // r3
    out = pl.pallas_call(
        _stage3_body,
        grid=(nb,),
        in_specs=[
            pl.BlockSpec((r3, half), lambda i: (i, 0)),
            pl.BlockSpec((r3, half), lambda i: (i + nb, 0)),
        ],
        out_specs=pl.BlockSpec((r3, d), lambda i: (i, 0)),
        out_shape=jax.ShapeDtypeStruct((n, d), jnp.float32),
    )(sup, sup)
    return out
